# baseline (speedup 1.0000x reference)
import jax
import jax.numpy as jnp
from jax import lax
from jax.experimental import pallas as pl
from jax.experimental.pallas import tpu as pltpu
from jax.experimental.pallas import tpu_sc as plsc

N = 10000
H = 2048
E = 320000
NNZ = 100000
NC = 2
NS = 16
NW = NC * NS
CHUNK = 128
ECH = 81
E_PAD = NW * ECH * CHUNK
HCH = 27
NNZ_PAD = NW * HCH * CHUNK
N_ACC = 128 * 79
H_ACC = 128 * 17

_f32 = jnp.float32
_mesh = lambda: plsc.VectorSubcoreMesh(core_axis_name="c", subcore_axis_name="s")


def _ceil(a, b):
    return -(-a // b)


def _make_segsum(n_acc, n_out, nchunks, gather, count):
    nzb = _ceil(n_acc // 128, NS)
    wb = _ceil(_ceil(n_out, NS), 16) * 16
    assert nchunks % 3 == 0

    def body(*refs):
        it = iter(refs)
        p = next(it) if gather else None
        src = next(it) if gather else None
        dst = next(it)
        zero_b = next(it) if gather else None
        zeroc_b = next(it) if count else None
        ones_b = next(it) if count else None
        out_s = next(it) if gather else None
        out_c = next(it) if count else None
        acc = next(it) if gather else None
        accc = next(it) if count else None
        isv = next(it) if gather else None
        idv = next(it)
        rows = next(it) if gather else None
        ones_v = next(it) if count else None
        gsem = [next(it) for _ in range(3)] if gather else None
        ssem = [next(it) for _ in range(3)] if gather else None
        csem = [next(it) for _ in range(3)] if count else None

        c = lax.axis_index("c")
        s = lax.axis_index("s")
        wid = c * NS + s
        ebase = wid * (nchunks * CHUNK)
        if count:
            pltpu.sync_copy(ones_b, ones_v)
        for i in range(nzb):
            off = jnp.minimum((s + NS * i) * 128, n_acc - 128)
            if gather:
                pltpu.sync_copy(zero_b, acc.at[pl.ds(off, 128)])
            if count:
                pltpu.sync_copy(zeroc_b, accc.at[pl.ds(off, 128)])
        plsc.subcore_barrier()

        if gather:
            def stage(kk, b):
                off = ebase + kk * CHUNK
                pltpu.sync_copy(src.at[pl.ds(off, CHUNK)], isv.at[b])
                pltpu.sync_copy(dst.at[pl.ds(off, CHUNK)], idv.at[b])
                pltpu.async_copy(p.at[isv.at[b]], rows.at[b], gsem[b])

            stage(0, 0)
            stage(1, 1)

            def triple(j, carry):
                for b in range(3):
                    kk = 3 * j + b
                    bp = (b + 2) % 3
                    pltpu.make_async_copy(p.at[isv.at[b]], rows.at[b],
                                          gsem[b]).wait()

                    @pl.when(kk >= 1)
                    def _():
                        pltpu.make_async_copy(rows.at[bp],
                                              acc.at[idv.at[bp]],
                                              ssem[bp]).wait()
                        if count:
                            pltpu.make_async_copy(ones_v, accc.at[idv.at[bp]],
                                                  csem[bp]).wait()

                    @pl.when(kk + 2 < nchunks)
                    def _():
                        stage(kk + 2, bp)

                    pltpu.async_copy(rows.at[b], acc.at[idv.at[b]],
                                     ssem[b], add=True)
                    if count:
                        pltpu.async_copy(ones_v, accc.at[idv.at[b]],
                                         csem[b], add=True)
                return carry

            lax.fori_loop(0, nchunks // 3, triple, 0)
            bl = (nchunks - 1) % 3
            pltpu.make_async_copy(rows.at[bl], acc.at[idv.at[bl]],
                                  ssem[bl]).wait()
            if count:
                pltpu.make_async_copy(ones_v, accc.at[idv.at[bl]],
                                      csem[bl]).wait()
        else:
            def triple(j, carry):
                for b in range(3):
                    kk = 3 * j + b

                    @pl.when(kk >= 3)
                    def _():
                        pltpu.make_async_copy(ones_v, accc.at[idv.at[b]],
                                              csem[b]).wait()

                    off = ebase + kk * CHUNK
                    pltpu.sync_copy(dst.at[pl.ds(off, CHUNK)], idv.at[b])
                    pltpu.async_copy(ones_v, accc.at[idv.at[b]],
                                     csem[b], add=True)
                return carry

            lax.fori_loop(0, nchunks // 3, triple, 0)
            for b in range(3):
                pltpu.make_async_copy(ones_v, accc.at[idv.at[b]],
                                      csem[b]).wait()
        plsc.subcore_barrier()

        wbo = jnp.minimum(s * wb, n_out - wb)
        if gather:
            pltpu.sync_copy(acc.at[pl.ds(wbo, wb)], out_s.at[c, pl.ds(wbo, wb)])
        if count:
            pltpu.sync_copy(accc.at[pl.ds(wbo, wb)], out_c.at[c, pl.ds(wbo, wb)])

    out_type = []
    if gather:
        out_type.append(jax.ShapeDtypeStruct((NC, n_out, 128), _f32))
    if count:
        out_type.append(jax.ShapeDtypeStruct((NC, n_out, 128), _f32))
    scratch = []
    if gather:
        scratch.append(pltpu.VMEM_SHARED((n_acc, 128), _f32))
    if count:
        scratch.append(pltpu.VMEM_SHARED((n_acc, 128), _f32))
    if gather:
        scratch.append(pltpu.VMEM((3, CHUNK), jnp.int32))
    scratch.append(pltpu.VMEM((3, CHUNK), jnp.int32))
    if gather:
        scratch.append(pltpu.VMEM((3, CHUNK, 128), _f32))
    if count:
        scratch.append(pltpu.VMEM((CHUNK, 128), _f32))
    if gather:
        scratch += [pltpu.SemaphoreType.DMA] * 6
    if count:
        scratch += [pltpu.SemaphoreType.DMA] * 3
    return pl.kernel(body, out_type=out_type, mesh=_mesh(), scratch_types=scratch)


_segsum = _make_segsum(N_ACC, N, ECH, True, False)
_deg = _make_segsum(N_ACC, N, ECH, False, True)
_segsum_hyper = _make_segsum(H_ACC, H, HCH, True, True)


_BM = 1000


_TW = 104


def _proj_body(attr, vis, tabs, ws, we, o):
    a = attr[...]
    acc = jnp.dot(vis[...], we[...], preferred_element_type=_f32)
    cols = jax.lax.broadcasted_iota(jnp.int32, (_BM, _TW), 1)
    for t in range(4):
        T = jnp.dot(tabs[t], ws[t], preferred_element_type=_f32)
        oh = (cols == a[:, t:t + 1]).astype(_f32)
        acc += jnp.dot(oh, T, preferred_element_type=_f32)
    o[...] = acc


def _proj(attr, vis, tabs, ws, we):
    g = N // _BM
    return pl.pallas_call(
        _proj_body,
        grid=(g,),
        in_specs=[pl.BlockSpec((_BM, 8), lambda i: (i, 0)),
                  pl.BlockSpec((_BM, 64), lambda i: (i, 0)),
                  pl.BlockSpec((4, _TW, 64), lambda i: (0, 0, 0)),
                  pl.BlockSpec((4, 64, 128), lambda i: (0, 0, 0)),
                  pl.BlockSpec((64, 128), lambda i: (0, 0))],
        out_specs=pl.BlockSpec((_BM, 128), lambda i: (i, 0)),
        out_shape=jax.ShapeDtypeStruct((N, 128), _f32),
    )(attr, vis, tabs, ws, we)


def _layer_body(S, Dg, p, W, b, o):
    agg = S[0] + S[1] + p[...]
    deg = (Dg[0, 0] + Dg[0, 1] + 1.0)[:, None]
    h = jnp.maximum(agg / deg + b[...], 0.0)
    o[...] = jnp.dot(h, W[...], preferred_element_type=_f32)


def _layer(S, Dg, p, W, b):
    g = N // _BM
    return pl.pallas_call(
        _layer_body,
        grid=(g,),
        in_specs=[pl.BlockSpec((NC, _BM, 128), lambda i: (0, i, 0)),
                  pl.BlockSpec((1, NC, _BM), lambda i: (i, 0, 0)),
                  pl.BlockSpec((_BM, 128), lambda i: (i, 0)),
                  pl.BlockSpec((128, 128), lambda i: (0, 0)),
                  pl.BlockSpec((1, 128), lambda i: (0, 0))],
        out_specs=pl.BlockSpec((_BM, 128), lambda i: (i, 0)),
        out_shape=jax.ShapeDtypeStruct((N, 128), _f32),
    )(S, Dg, p, W, b)


def _layer2_body(S, Dg, p, W, b, oh, op):
    agg = S[0] + S[1] + p[...]
    deg = (Dg[0, 0] + Dg[0, 1] + 1.0)[:, None]
    h = jnp.maximum(agg / deg + b[...], 0.0)
    oh[...] = h
    op[...] = jnp.dot(h, W[...], preferred_element_type=_f32)


def _layer2(S, Dg, p, W, b):
    g = N // _BM
    return pl.pallas_call(
        _layer2_body,
        grid=(g,),
        in_specs=[pl.BlockSpec((NC, _BM, 128), lambda i: (0, i, 0)),
                  pl.BlockSpec((1, NC, _BM), lambda i: (i, 0, 0)),
                  pl.BlockSpec((_BM, 128), lambda i: (i, 0)),
                  pl.BlockSpec((128, 128), lambda i: (0, 0)),
                  pl.BlockSpec((1, 128), lambda i: (0, 0))],
        out_specs=[pl.BlockSpec((_BM, 128), lambda i: (i, 0)),
                   pl.BlockSpec((_BM, 128), lambda i: (i, 0))],
        out_shape=[jax.ShapeDtypeStruct((N, 128), _f32),
                   jax.ShapeDtypeStruct((N, 128), _f32)],
    )(S, Dg, p, W, b)


def _hyper_body(S, C, b, o):
    cnt = (C[0] + C[1])[:, None]
    r = 1.0 / jnp.maximum(cnt, 1.0)
    o[...] = jnp.maximum((S[0] + S[1]) * r + b[...], 0.0)


def _hyper(S, C, b):
    return pl.pallas_call(
        _hyper_body,
        grid=(1,),
        in_specs=[pl.BlockSpec((NC, H, 128), lambda i: (0, 0, 0)),
                  pl.BlockSpec((NC, H), lambda i: (0, 0)),
                  pl.BlockSpec((1, 128), lambda i: (0, 0))],
        out_specs=pl.BlockSpec((H, 128), lambda i: (0, 0)),
        out_shape=jax.ShapeDtypeStruct((H, 128), _f32),
    )(S, C, b)


def kernel(seg_attr, seg_vis_feat, edge_index, hyperedge_index, num_nodes,
           num_hyperedges, id_table, len_table, lng_table, lat_table,
           W1, b1, W2, b2, Wh, bh):
    i32 = jnp.int32
    src = edge_index[0].astype(i32)
    dst = edge_index[1].astype(i32)
    hsrc = hyperedge_index[0].astype(i32)
    hdst = hyperedge_index[1].astype(i32)

    pe = E_PAD - E
    pi = jnp.arange(pe, dtype=i32)
    src_p = jnp.concatenate([src, pi % N])
    dst_p = jnp.concatenate([dst, N + (pi % (N_ACC - N))])
    ph_ = NNZ_PAD - NNZ
    hpi = jnp.arange(ph_, dtype=i32)
    hsrc_p = jnp.concatenate([hsrc, hpi % N])
    hdst_p = jnp.concatenate([hdst, H + (hpi % (H_ACC - H))])

    zero128 = jnp.zeros((128, 128), _f32)
    zeroc = jnp.zeros((128, 128), _f32)
    onesc = jnp.zeros((128, 128), _f32).at[:, 0].set(1.0)

    attr8 = jnp.pad(seg_attr.astype(i32), ((0, 0), (0, 4)))

    def padt(t):
        return jnp.pad(t, ((0, _TW - t.shape[0]), (0, 64 - t.shape[1])))

    def padw(w):
        return jnp.pad(w, ((0, 64 - w.shape[0]), (0, 0)))

    tabs4 = jnp.stack([padt(id_table[:_TW]), padt(len_table), padt(lng_table),
                       padt(lat_table)])
    ws4 = jnp.stack([W1[:64], padw(W1[64:80]), padw(W1[80:96]),
                     padw(W1[96:112])])
    (D1,) = _deg(dst_p, zeroc, onesc)
    p1 = _proj(attr8, seg_vis_feat, tabs4, ws4, W1[112:176])
    (S1,) = _segsum(p1, src_p, dst_p, zero128)
    D1c = D1[:, :, 0].reshape(NC, N // _BM, _BM).transpose(1, 0, 2)
    p2 = _layer(S1, D1c, p1, W2, b1.reshape(1, 128))
    (S2,) = _segsum(p2, src_p, dst_p, zero128)
    seg_h, ph = _layer2(S2, D1c, p2, Wh, b2.reshape(1, 128))
    Sh, Ch = _segsum_hyper(ph, hsrc_p, hdst_p, zero128, zeroc, onesc)
    tra_h = _hyper(Sh, Ch[:, :, 0], bh.reshape(1, 128))
    return seg_h, tra_h


# speedup vs baseline: 1.0010x; 1.0010x over previous
import jax
import jax.numpy as jnp
from jax import lax
from jax.experimental import pallas as pl
from jax.experimental.pallas import tpu as pltpu
from jax.experimental.pallas import tpu_sc as plsc

N = 10000
H = 2048
E = 320000
NNZ = 100000
NC = 2
NS = 16
NW = NC * NS
CHUNK = 128
ECH = 81
E_PAD = NW * ECH * CHUNK
HCH = 27
NNZ_PAD = NW * HCH * CHUNK
N_ACC = 128 * 79
H_ACC = 128 * 17

_f32 = jnp.float32
_mesh = lambda: plsc.VectorSubcoreMesh(core_axis_name="c", subcore_axis_name="s")


def _ceil(a, b):
    return -(-a // b)


def _make_segsum(n_acc, n_out, nchunks, gather, count, chunk=CHUNK):
    nzb = _ceil(n_acc // 128, NS)
    wb = _ceil(_ceil(n_out, NS), 16) * 16
    assert nchunks % 3 == 0
    CH = chunk

    def body(*refs):
        it = iter(refs)
        p = next(it) if gather else None
        src = next(it) if gather else None
        dst = next(it)
        zero_b = next(it) if gather else None
        zeroc_b = next(it) if count else None
        ones_b = next(it) if count else None
        out_s = next(it) if gather else None
        out_c = next(it) if count else None
        acc = next(it) if gather else None
        accc = next(it) if count else None
        isv = next(it) if gather else None
        idv = next(it)
        rows = next(it) if gather else None
        ones_v = next(it) if count else None
        gsem = [next(it) for _ in range(3)] if gather else None
        ssem = [next(it) for _ in range(3)] if gather else None
        csem = [next(it) for _ in range(3)] if count else None

        c = lax.axis_index("c")
        s = lax.axis_index("s")
        wid = c * NS + s
        ebase = wid * (nchunks * CH)
        if count:
            pltpu.sync_copy(ones_b, ones_v)
        for i in range(nzb):
            off = jnp.minimum((s + NS * i) * 128, n_acc - 128)
            if gather:
                pltpu.sync_copy(zero_b, acc.at[pl.ds(off, 128)])
            if count:
                pltpu.sync_copy(zeroc_b, accc.at[pl.ds(off, 128)])
        plsc.subcore_barrier()

        if gather:
            def stage(kk, b):
                off = ebase + kk * CH
                pltpu.sync_copy(src.at[pl.ds(off, CH)], isv.at[b])
                pltpu.sync_copy(dst.at[pl.ds(off, CH)], idv.at[b])
                pltpu.async_copy(p.at[isv.at[b]], rows.at[b], gsem[b])

            stage(0, 0)
            stage(1, 1)

            def triple(j, carry):
                for b in range(3):
                    kk = 3 * j + b
                    bp = (b + 2) % 3
                    pltpu.make_async_copy(p.at[isv.at[b]], rows.at[b],
                                          gsem[b]).wait()

                    @pl.when(kk >= 1)
                    def _():
                        pltpu.make_async_copy(rows.at[bp],
                                              acc.at[idv.at[bp]],
                                              ssem[bp]).wait()
                        if count:
                            pltpu.make_async_copy(ones_v, accc.at[idv.at[bp]],
                                                  csem[bp]).wait()

                    @pl.when(kk + 2 < nchunks)
                    def _():
                        stage(kk + 2, bp)

                    pltpu.async_copy(rows.at[b], acc.at[idv.at[b]],
                                     ssem[b], add=True)
                    if count:
                        pltpu.async_copy(ones_v, accc.at[idv.at[b]],
                                         csem[b], add=True)
                return carry

            lax.fori_loop(0, nchunks // 3, triple, 0)
            bl = (nchunks - 1) % 3
            pltpu.make_async_copy(rows.at[bl], acc.at[idv.at[bl]],
                                  ssem[bl]).wait()
            if count:
                pltpu.make_async_copy(ones_v, accc.at[idv.at[bl]],
                                      csem[bl]).wait()
        else:
            def triple(j, carry):
                for b in range(3):
                    kk = 3 * j + b

                    @pl.when(kk >= 3)
                    def _():
                        pltpu.make_async_copy(ones_v, accc.at[idv.at[b]],
                                              csem[b]).wait()

                    off = ebase + kk * CH
                    pltpu.sync_copy(dst.at[pl.ds(off, CH)], idv.at[b])
                    pltpu.async_copy(ones_v, accc.at[idv.at[b]],
                                     csem[b], add=True)
                return carry

            lax.fori_loop(0, nchunks // 3, triple, 0)
            for b in range(3):
                pltpu.make_async_copy(ones_v, accc.at[idv.at[b]],
                                      csem[b]).wait()
        plsc.subcore_barrier()

        wbo = jnp.minimum(s * wb, n_out - wb)
        if gather:
            pltpu.sync_copy(acc.at[pl.ds(wbo, wb)], out_s.at[c, pl.ds(wbo, wb)])
        if count:
            pltpu.sync_copy(accc.at[pl.ds(wbo, wb)], out_c.at[c, pl.ds(wbo, wb)])

    out_type = []
    if gather:
        out_type.append(jax.ShapeDtypeStruct((NC, n_out, 128), _f32))
    if count:
        out_type.append(jax.ShapeDtypeStruct((NC, n_out, 128), _f32))
    scratch = []
    if gather:
        scratch.append(pltpu.VMEM_SHARED((n_acc, 128), _f32))
    if count:
        scratch.append(pltpu.VMEM_SHARED((n_acc, 128), _f32))
    if gather:
        scratch.append(pltpu.VMEM((3, CH), jnp.int32))
    scratch.append(pltpu.VMEM((3, CH), jnp.int32))
    if gather:
        scratch.append(pltpu.VMEM((3, CH, 128), _f32))
    if count:
        scratch.append(pltpu.VMEM((CH, 128), _f32))
    if gather:
        scratch += [pltpu.SemaphoreType.DMA] * 6
    if count:
        scratch += [pltpu.SemaphoreType.DMA] * 3
    return pl.kernel(body, out_type=out_type, mesh=_mesh(), scratch_types=scratch)


_segsum = _make_segsum(N_ACC, N, ECH, True, False)
_deg = _make_segsum(N_ACC, N, ECH, False, True)
_segsum_hyper = _make_segsum(H_ACC, H, HCH, True, True)


_BM = 1000


_TW = 104


def _proj_body(attr, vis, tabs, ws, we, o):
    a = attr[...]
    acc = jnp.dot(vis[...], we[...], preferred_element_type=_f32)
    cols = jax.lax.broadcasted_iota(jnp.int32, (_BM, _TW), 1)
    for t in range(4):
        T = jnp.dot(tabs[t], ws[t], preferred_element_type=_f32)
        oh = (cols == a[:, t:t + 1]).astype(_f32)
        acc += jnp.dot(oh, T, preferred_element_type=_f32)
    o[...] = acc


def _proj(attr, vis, tabs, ws, we):
    g = N // _BM
    return pl.pallas_call(
        _proj_body,
        grid=(g,),
        in_specs=[pl.BlockSpec((_BM, 8), lambda i: (i, 0)),
                  pl.BlockSpec((_BM, 64), lambda i: (i, 0)),
                  pl.BlockSpec((4, _TW, 64), lambda i: (0, 0, 0)),
                  pl.BlockSpec((4, 64, 128), lambda i: (0, 0, 0)),
                  pl.BlockSpec((64, 128), lambda i: (0, 0))],
        out_specs=pl.BlockSpec((_BM, 128), lambda i: (i, 0)),
        out_shape=jax.ShapeDtypeStruct((N, 128), _f32),
    )(attr, vis, tabs, ws, we)


def _layer_body(S, Dg, p, W, b, o):
    agg = S[0] + S[1] + p[...]
    deg = (Dg[0, 0] + Dg[0, 1] + 1.0)[:, None]
    h = jnp.maximum(agg / deg + b[...], 0.0)
    o[...] = jnp.dot(h, W[...], preferred_element_type=_f32)


def _layer(S, Dg, p, W, b):
    g = N // _BM
    return pl.pallas_call(
        _layer_body,
        grid=(g,),
        in_specs=[pl.BlockSpec((NC, _BM, 128), lambda i: (0, i, 0)),
                  pl.BlockSpec((1, NC, _BM), lambda i: (i, 0, 0)),
                  pl.BlockSpec((_BM, 128), lambda i: (i, 0)),
                  pl.BlockSpec((128, 128), lambda i: (0, 0)),
                  pl.BlockSpec((1, 128), lambda i: (0, 0))],
        out_specs=pl.BlockSpec((_BM, 128), lambda i: (i, 0)),
        out_shape=jax.ShapeDtypeStruct((N, 128), _f32),
    )(S, Dg, p, W, b)


def _layer2_body(S, Dg, p, W, b, oh, op):
    agg = S[0] + S[1] + p[...]
    deg = (Dg[0, 0] + Dg[0, 1] + 1.0)[:, None]
    h = jnp.maximum(agg / deg + b[...], 0.0)
    oh[...] = h
    op[...] = jnp.dot(h, W[...], preferred_element_type=_f32)


def _layer2(S, Dg, p, W, b):
    g = N // _BM
    return pl.pallas_call(
        _layer2_body,
        grid=(g,),
        in_specs=[pl.BlockSpec((NC, _BM, 128), lambda i: (0, i, 0)),
                  pl.BlockSpec((1, NC, _BM), lambda i: (i, 0, 0)),
                  pl.BlockSpec((_BM, 128), lambda i: (i, 0)),
                  pl.BlockSpec((128, 128), lambda i: (0, 0)),
                  pl.BlockSpec((1, 128), lambda i: (0, 0))],
        out_specs=[pl.BlockSpec((_BM, 128), lambda i: (i, 0)),
                   pl.BlockSpec((_BM, 128), lambda i: (i, 0))],
        out_shape=[jax.ShapeDtypeStruct((N, 128), _f32),
                   jax.ShapeDtypeStruct((N, 128), _f32)],
    )(S, Dg, p, W, b)


def _hyper_body(S, C, b, o):
    cnt = (C[0] + C[1])[:, None]
    r = 1.0 / jnp.maximum(cnt, 1.0)
    o[...] = jnp.maximum((S[0] + S[1]) * r + b[...], 0.0)


def _hyper(S, C, b):
    return pl.pallas_call(
        _hyper_body,
        grid=(1,),
        in_specs=[pl.BlockSpec((NC, H, 128), lambda i: (0, 0, 0)),
                  pl.BlockSpec((NC, H), lambda i: (0, 0)),
                  pl.BlockSpec((1, 128), lambda i: (0, 0))],
        out_specs=pl.BlockSpec((H, 128), lambda i: (0, 0)),
        out_shape=jax.ShapeDtypeStruct((H, 128), _f32),
    )(S, C, b)


def kernel(seg_attr, seg_vis_feat, edge_index, hyperedge_index, num_nodes,
           num_hyperedges, id_table, len_table, lng_table, lat_table,
           W1, b1, W2, b2, Wh, bh):
    i32 = jnp.int32
    src = edge_index[0].astype(i32)
    dst = edge_index[1].astype(i32)
    hsrc = hyperedge_index[0].astype(i32)
    hdst = hyperedge_index[1].astype(i32)

    pe = E_PAD - E
    pi = jnp.arange(pe, dtype=i32)
    src_p = jnp.concatenate([src, pi % N])
    dst_p = jnp.concatenate([dst, N + (pi % (N_ACC - N))])
    ph_ = NNZ_PAD - NNZ
    hpi = jnp.arange(ph_, dtype=i32)
    hsrc_p = jnp.concatenate([hsrc, hpi % N])
    hdst_p = jnp.concatenate([hdst, H + (hpi % (H_ACC - H))])

    zero128 = jnp.zeros((128, 128), _f32)
    zeroc = jnp.zeros((128, 128), _f32)
    onesc = jnp.zeros((128, 128), _f32).at[:, 0].set(1.0)

    attr8 = jnp.pad(seg_attr.astype(i32), ((0, 0), (0, 4)))

    def padt(t):
        return jnp.pad(t, ((0, _TW - t.shape[0]), (0, 64 - t.shape[1])))

    def padw(w):
        return jnp.pad(w, ((0, 64 - w.shape[0]), (0, 0)))

    tabs4 = jnp.stack([padt(id_table[:_TW]), padt(len_table), padt(lng_table),
                       padt(lat_table)])
    ws4 = jnp.stack([W1[:64], padw(W1[64:80]), padw(W1[80:96]),
                     padw(W1[96:112])])
    (D1,) = _deg(dst_p, zeroc, onesc)
    p1 = _proj(attr8, seg_vis_feat, tabs4, ws4, W1[112:176])
    (S1,) = _segsum(p1, src_p, dst_p, zero128)
    D1c = D1[:, :, 0].reshape(NC, N // _BM, _BM).transpose(1, 0, 2)
    p2 = _layer(S1, D1c, p1, W2, b1.reshape(1, 128))
    (S2,) = _segsum(p2, src_p, dst_p, zero128)
    seg_h, ph = _layer2(S2, D1c, p2, Wh, b2.reshape(1, 128))
    Sh, Ch = _segsum_hyper(ph, hsrc_p, hdst_p, zero128, zeroc, onesc)
    tra_h = _hyper(Sh, Ch[:, :, 0], bh.reshape(1, 128))
    return seg_h, tra_h
